# Initial kernel scaffold; baseline (speedup 1.0000x reference)
import jax
import jax.numpy as jnp
from jax.experimental import pallas as pl


def kernel(x, edge_index, pos_edge_index, neg_edge_index, Wl1, bl1, Wr1, g1, b1, Wl2, bl2, Wr2, g2, b2, Wl3, bl3, Wr3, Wp1, bp1, Wp2, bp2, Wc1, bc1, Wc2, bc2):
    raise NotImplementedError("write your pallas kernel here")


# speedup vs baseline: 1.1077x; 1.1077x over previous
import numpy as np
import jax
import jax.numpy as jnp
from jax.experimental import pallas as pl
from jax.experimental.pallas import tpu as pltpu

_N = 10000
_E = 320000
_P2 = 131072
_BN_S = np.float32(1.0 / np.sqrt(1.0 + 1e-5))


def _phase_a(x, agg1, cnt, Wl1T, bl1, Wr1T, g1, b1):
    def body(x_ref, agg_ref, cnt_ref, wl_ref, blr, wr_ref, gr, br, ha, hb):
        cnt = jnp.maximum(cnt_ref[...], 1.0)
        mean = agg_ref[...] / cnt
        h = jnp.dot(mean, wl_ref[...], preferred_element_type=jnp.float32)
        h = h + jnp.dot(x_ref[...], wr_ref[...], preferred_element_type=jnp.float32)
        h = (h + blr[...]) * (gr[...] * _BN_S) + br[...]
        h = jnp.maximum(h, 0.0)
        ha[...] = h[:, :128]
        hb[...] = h[:, 128:]

    return pl.pallas_call(
        body,
        out_shape=(jax.ShapeDtypeStruct((_N, 128), jnp.float32),
                   jax.ShapeDtypeStruct((_N, 128), jnp.float32)),
    )(x, agg1, cnt, Wl1T, bl1, Wr1T, g1, b1)


def _phase_b(h1a, h1b, agg2a, agg2b, cnt, Wl2T, bl2, Wr2T, g2, b2, Wl3T, Wr3T):
    def body(ha, hb, aa, ab, cnt_ref, wl, blr, wr, gr, br, wl3, wr3, yl_ref, yr_ref):
        cnt = jnp.maximum(cnt_ref[...], 1.0)
        mean = jnp.concatenate([aa[...] / cnt, ab[...] / cnt], axis=-1)
        h1 = jnp.concatenate([ha[...], hb[...]], axis=-1)
        h = jnp.dot(mean, wl[...], preferred_element_type=jnp.float32)
        h = h + jnp.dot(h1, wr[...], preferred_element_type=jnp.float32)
        h = (h + blr[...]) * (gr[...] * _BN_S) + br[...]
        h = jnp.maximum(h, 0.0)
        yl_ref[...] = jnp.dot(h, wl3[...], preferred_element_type=jnp.float32)
        yr_ref[...] = jnp.dot(h, wr3[...], preferred_element_type=jnp.float32)

    return pl.pallas_call(
        body,
        out_shape=(jax.ShapeDtypeStruct((_N, 128), jnp.float32),
                   jax.ShapeDtypeStruct((_N, 128), jnp.float32)),
    )(h1a, h1b, agg2a, agg2b, cnt, Wl2T, bl2, Wr2T, g2, b2, Wl3T, Wr3T)


def _phase_c(agg3, cnt, yr, bl3, Wc1T, bc1, Wc2T, bc2, Wp1aT, Wp1bT, bp1):
    def body(agg_ref, cnt_ref, yr_ref, bl3r, wc1, bc1r, wc2, bc2r, wpa, wpb,
             bp1r, z_ref, nl_ref, zu_ref, zv_ref):
        cnt = jnp.maximum(cnt_ref[...], 1.0)
        z = agg_ref[...] / cnt + bl3r[...] + yr_ref[...]
        z_ref[...] = z
        t = jnp.maximum(jnp.dot(z, wc1[...], preferred_element_type=jnp.float32)
                        + bc1r[...], 0.0)
        nl_ref[...] = jnp.dot(t, wc2[...], preferred_element_type=jnp.float32) + bc2r[...]
        zu_ref[...] = jnp.dot(z, wpa[...], preferred_element_type=jnp.float32) + bp1r[...]
        zv_ref[...] = jnp.dot(z, wpb[...], preferred_element_type=jnp.float32)

    return pl.pallas_call(
        body,
        out_shape=(jax.ShapeDtypeStruct((_N, 128), jnp.float32),
                   jax.ShapeDtypeStruct((_N, 4), jnp.float32),
                   jax.ShapeDtypeStruct((_N, 128), jnp.float32),
                   jax.ShapeDtypeStruct((_N, 128), jnp.float32)),
    )(agg3, cnt, yr, bl3, Wc1T, bc1, Wc2T, bc2, Wp1aT, Wp1bT, bp1)


def _phase_d(gu, gv, wp2, bp2):
    _NB = 16
    _B = _P2 // _NB

    def body(gu_ref, gv_ref, w_ref, b_ref, out_ref):
        e = jnp.maximum(gu_ref[...] + gv_ref[...], 0.0)
        logit = jnp.sum(e * w_ref[...], axis=-1) + b_ref[0, 0]
        out_ref[...] = jax.nn.sigmoid(logit).reshape(_B // 128, 128)

    out = pl.pallas_call(
        body,
        grid=(_NB,),
        in_specs=[pl.BlockSpec((_B, 128), lambda i: (i, 0)),
                  pl.BlockSpec((_B, 128), lambda i: (i, 0)),
                  pl.BlockSpec((1, 128), lambda i: (0, 0)),
                  pl.BlockSpec((1, 1), lambda i: (0, 0))],
        out_specs=pl.BlockSpec((_B // 128, 128), lambda i: (i, 0)),
        out_shape=jax.ShapeDtypeStruct((_P2 // 128, 128), jnp.float32),
    )(gu, gv, wp2, bp2)
    return out.reshape(_P2)


def kernel(x, edge_index, pos_edge_index, neg_edge_index, Wl1, bl1, Wr1, g1, b1,
           Wl2, bl2, Wr2, g2, b2, Wl3, bl3, Wr3, Wp1, bp1, Wp2, bp2, Wc1, bc1,
           Wc2, bc2):
    src = edge_index[0]
    dst = edge_index[1]

    ones = jnp.ones((_E,), jnp.float32)
    cnt = jax.ops.segment_sum(ones, dst, num_segments=_N).reshape(_N, 1)
    agg1 = jax.ops.segment_sum(jnp.take(x, src, axis=0), dst, num_segments=_N)

    h1a, h1b = _phase_a(
        x, agg1, cnt, Wl1.T, bl1.reshape(1, -1), Wr1.T, g1.reshape(1, -1),
        b1.reshape(1, -1))

    h1 = jnp.concatenate([h1a, h1b], axis=-1)
    agg2 = jax.ops.segment_sum(jnp.take(h1, src, axis=0), dst, num_segments=_N)

    yl, yr = _phase_b(
        h1a, h1b, agg2[:, :128], agg2[:, 128:], cnt, Wl2.T, bl2.reshape(1, -1),
        Wr2.T, g2.reshape(1, -1), b2.reshape(1, -1), Wl3.T, Wr3.T)

    agg3 = jax.ops.segment_sum(jnp.take(yl, src, axis=0), dst, num_segments=_N)

    z, node_logits, zu_proj, zv_proj = _phase_c(
        agg3, cnt, yr, bl3.reshape(1, -1), Wc1.T, bc1.reshape(1, -1), Wc2.T,
        bc2.reshape(1, -1), Wp1[:, :128].T, Wp1[:, 128:].T, bp1.reshape(1, -1))

    u = jnp.concatenate([pos_edge_index[0], neg_edge_index[0]])
    v = jnp.concatenate([pos_edge_index[1], neg_edge_index[1]])
    gu = jnp.take(zu_proj, u, axis=0)
    gv = jnp.take(zv_proj, v, axis=0)

    link_probs = _phase_d(gu, gv, Wp2, bp2.reshape(1, 1))
    return (z, node_logits, link_probs)


# speedup vs baseline: 3.6646x; 3.3084x over previous
import numpy as np
import jax
from jax import lax
import jax.numpy as jnp
from jax.experimental import pallas as pl
from jax.experimental.pallas import tpu as pltpu
from jax.experimental.pallas import tpu_sc as plsc

_N = 10000
_E = 320000
_P2 = 131072
_BN_S = np.float32(1.0 / np.sqrt(1.0 + 1e-5))

_NC, _NS = 2, 16
_NPAD = 10240
_EPAD = 327680
_CHUNK = 256
_RPT = _NPAD // _NS

_MESH = plsc.VectorSubcoreMesh(core_axis_name="c", subcore_axis_name="s",
                               num_cores=_NC, num_subcores=_NS)


def _sc_seg_sum(ta, tb, srcp, dstp, zrows, zcnt, ones_r, with_cnt):
    ecp_t = _EPAD // _NS
    nchunks = ecp_t // _CHUNK

    def body(tar, tbr, srcr, dstr, zr, zc, onesr, out, outc,
             idx_s, idx_d, rows, ones_v, acc, accc, sem):
        c = lax.axis_index("c")
        s = lax.axis_index("s")
        r0 = s * _RPT
        pltpu.sync_copy(zr, acc.at[pl.ds(r0, _RPT)])
        if with_cnt:
            @pl.when(c == 0)
            def _():
                pltpu.sync_copy(zc, accc.at[pl.ds(r0, _RPT)])
                pltpu.sync_copy(onesr, ones_v)
        plsc.subcore_barrier()
        base = s * ecp_t

        @pl.loop(0, nchunks)
        def _(k):
            off = base + k * _CHUNK
            pltpu.sync_copy(srcr.at[pl.ds(off, _CHUNK)], idx_s)
            pltpu.sync_copy(dstr.at[pl.ds(off, _CHUNK)], idx_d)

            @pl.when(c == 0)
            def _():
                pltpu.async_copy(tar.at[idx_s], rows, sem).wait()

            @pl.when(c == 1)
            def _():
                pltpu.async_copy(tbr.at[idx_s], rows, sem).wait()

            pltpu.sync_copy(rows, acc.at[idx_d], add=True)
            if with_cnt:
                @pl.when(c == 0)
                def _():
                    pltpu.sync_copy(ones_v, accc.at[idx_d], add=True)

        plsc.subcore_barrier()
        pltpu.sync_copy(acc.at[pl.ds(r0, _RPT)], out.at[c, pl.ds(r0, _RPT)])
        if with_cnt:
            @pl.when(c == 0)
            def _():
                pltpu.sync_copy(accc.at[pl.ds(r0, _RPT)], outc.at[pl.ds(r0, _RPT)])

    f = pl.kernel(
        body,
        out_type=(jax.ShapeDtypeStruct((_NC, _NPAD, 64), jnp.float32),
                  jax.ShapeDtypeStruct((_NPAD, 16), jnp.float32)),
        mesh=_MESH,
        compiler_params=pltpu.CompilerParams(use_tc_tiling_on_sc=False),
        scratch_types=[
            pltpu.VMEM((_CHUNK,), jnp.int32),
            pltpu.VMEM((_CHUNK,), jnp.int32),
            pltpu.VMEM((_CHUNK, 64), jnp.float32),
            pltpu.VMEM((_CHUNK, 16), jnp.float32),
            pltpu.VMEM_SHARED((_NPAD, 64), jnp.float32),
            pltpu.VMEM_SHARED((_NPAD, 16), jnp.float32),
            pltpu.SemaphoreType.DMA,
        ],
    )
    return f(ta, tb, srcp, dstp, zrows, zcnt, ones_r)


def _sc_link_gather(zu, zv, uidx, vidx):
    ept = _P2 // (_NC * _NS)
    nch = ept // _CHUNK

    def body(tu, tv, ur, vr, gu, gv, idx_u, idx_v, rows_u, rows_v, sem):
        c = lax.axis_index("c")
        s = lax.axis_index("s")
        base = (c * _NS + s) * ept

        @pl.loop(0, nch)
        def _(k):
            off = base + k * _CHUNK
            pltpu.sync_copy(ur.at[pl.ds(off, _CHUNK)], idx_u)
            pltpu.sync_copy(vr.at[pl.ds(off, _CHUNK)], idx_v)
            pltpu.async_copy(tu.at[idx_u], rows_u, sem).wait()
            pltpu.sync_copy(rows_u, gu.at[pl.ds(off, _CHUNK)])
            pltpu.async_copy(tv.at[idx_v], rows_v, sem).wait()
            pltpu.sync_copy(rows_v, gv.at[pl.ds(off, _CHUNK)])

    f = pl.kernel(
        body,
        out_type=(jax.ShapeDtypeStruct((_P2, 128), jnp.float32),
                  jax.ShapeDtypeStruct((_P2, 128), jnp.float32)),
        mesh=_MESH,
        scratch_types=[
            pltpu.VMEM((_CHUNK,), jnp.int32),
            pltpu.VMEM((_CHUNK,), jnp.int32),
            pltpu.VMEM((_CHUNK, 128), jnp.float32),
            pltpu.VMEM((_CHUNK, 128), jnp.float32),
            pltpu.SemaphoreType.DMA,
        ],
    )
    return f(zu, zv, uidx, vidx)


def _cnt_col(craw):
    return jnp.maximum(craw[:, :1], 1.0)


_RB = 2000
_NRB = _N // _RB


def _row_call(body, n_rowed, weight_shapes, out_cols):
    def block(c):
        return pl.BlockSpec((_RB, c), lambda i: (i, 0))

    def full(shape):
        return pl.BlockSpec(shape, lambda i: (0, 0))

    def make(*arrays):
        in_specs = [block(a.shape[1]) for a in arrays[:n_rowed]]
        in_specs += [full(s) for s in weight_shapes]
        return pl.pallas_call(
            body,
            grid=(_NRB,),
            in_specs=in_specs,
            out_specs=tuple(block(c) for c in out_cols),
            out_shape=tuple(jax.ShapeDtypeStruct((_N, c), jnp.float32)
                            for c in out_cols),
        )(*arrays)
    return make


def _phase_a(x, a0, a1, craw, Wl1T, bl1, Wr1T, g1, b1):
    def body(x_ref, a0r, a1r, cr, wl_ref, blr, wr_ref, gr, br, h0, h1r, h2, h3):
        agg = jnp.concatenate([a0r[...], a1r[...]], axis=-1)
        mean = agg / _cnt_col(cr[...])
        h = jnp.dot(mean, wl_ref[...], preferred_element_type=jnp.float32)
        h = h + jnp.dot(x_ref[...], wr_ref[...], preferred_element_type=jnp.float32)
        h = (h + blr[...]) * (gr[...] * _BN_S) + br[...]
        h = jnp.maximum(h, 0.0)
        h0[...] = h[:, 0:64]
        h1r[...] = h[:, 64:128]
        h2[...] = h[:, 128:192]
        h3[...] = h[:, 192:256]

    wts = [Wl1T, bl1, Wr1T, g1, b1]
    return _row_call(body, 4, [w.shape for w in wts], (64, 64, 64, 64))(
        x, a0, a1, craw, *wts)


def _phase_b(hq, a0, a1, a2, a3, craw, Wl2T, bl2, Wr2T, g2, b2, Wl3T, Wr3T):
    def body(h0, h1r, h2r, h3, a0r, a1r, a2r, a3r, cr, wl, blr, wr, gr, br,
             wl3, wr3, yla_ref, ylb_ref, yr_ref):
        cnt = _cnt_col(cr[...])
        mean = jnp.concatenate(
            [a0r[...], a1r[...], a2r[...], a3r[...]], axis=-1) / cnt
        h1 = jnp.concatenate([h0[...], h1r[...], h2r[...], h3[...]], axis=-1)
        h = jnp.dot(mean, wl[...], preferred_element_type=jnp.float32)
        h = h + jnp.dot(h1, wr[...], preferred_element_type=jnp.float32)
        h = (h + blr[...]) * (gr[...] * _BN_S) + br[...]
        h = jnp.maximum(h, 0.0)
        yl = jnp.dot(h, wl3[...], preferred_element_type=jnp.float32)
        yla_ref[...] = yl[:, :64]
        ylb_ref[...] = yl[:, 64:]
        yr_ref[...] = jnp.dot(h, wr3[...], preferred_element_type=jnp.float32)

    wts = [Wl2T, bl2, Wr2T, g2, b2, Wl3T, Wr3T]
    return _row_call(body, 9, [w.shape for w in wts], (64, 64, 128))(
        *hq, a0, a1, a2, a3, craw, *wts)


def _phase_c(a0, a1, craw, yr, bl3, Wc1T, bc1, Wc2T, bc2, Wp1aT, Wp1bT, bp1):
    def body(a0r, a1r, cr, yr_ref, bl3r, wc1, bc1r, wc2, bc2r, wpa, wpb,
             bp1r, z_ref, nl_ref, zu_ref, zv_ref):
        agg = jnp.concatenate([a0r[...], a1r[...]], axis=-1)
        z = agg / _cnt_col(cr[...]) + bl3r[...] + yr_ref[...]
        z_ref[...] = z
        t = jnp.maximum(jnp.dot(z, wc1[...], preferred_element_type=jnp.float32)
                        + bc1r[...], 0.0)
        nl_ref[...] = jnp.dot(t, wc2[...], preferred_element_type=jnp.float32) + bc2r[...]
        zu_ref[...] = jnp.dot(z, wpa[...], preferred_element_type=jnp.float32) + bp1r[...]
        zv_ref[...] = jnp.dot(z, wpb[...], preferred_element_type=jnp.float32)

    wts = [bl3, Wc1T, bc1, Wc2T, bc2, Wp1aT, Wp1bT, bp1]
    return _row_call(body, 4, [w.shape for w in wts], (128, 4, 128, 128))(
        a0, a1, craw, yr, *wts)


def _phase_d(gu, gv, wp2, bp2):
    _NB = 16
    _B = _P2 // _NB

    def body(gu_ref, gv_ref, w_ref, b_ref, out_ref):
        e = jnp.maximum(gu_ref[...] + gv_ref[...], 0.0)
        logit = jnp.sum(e * w_ref[...], axis=-1) + b_ref[0, 0]
        out_ref[...] = jax.nn.sigmoid(logit).reshape(_B // 128, 128)

    out = pl.pallas_call(
        body,
        grid=(_NB,),
        in_specs=[pl.BlockSpec((_B, 128), lambda i: (i, 0)),
                  pl.BlockSpec((_B, 128), lambda i: (i, 0)),
                  pl.BlockSpec((1, 128), lambda i: (0, 0)),
                  pl.BlockSpec((1, 1), lambda i: (0, 0))],
        out_specs=pl.BlockSpec((_B // 128, 128), lambda i: (i, 0)),
        out_shape=jax.ShapeDtypeStruct((_P2 // 128, 128), jnp.float32),
    )(gu, gv, wp2, bp2)
    return out.reshape(_P2)


def kernel(x, edge_index, pos_edge_index, neg_edge_index, Wl1, bl1, Wr1, g1, b1,
           Wl2, bl2, Wr2, g2, b2, Wl3, bl3, Wr3, Wp1, bp1, Wp2, bp2, Wc1, bc1,
           Wc2, bc2):
    src = edge_index[0]
    dst = edge_index[1]
    srcp = jnp.concatenate([src, jnp.zeros((_EPAD - _E,), jnp.int32)])
    dstp = jnp.concatenate([dst, jnp.full((_EPAD - _E,), _NPAD - 1, jnp.int32)])

    zrows = jnp.zeros((_RPT, 64), jnp.float32)
    zcnt = jnp.zeros((_RPT, 16), jnp.float32)
    ones_r = jnp.ones((_CHUNK, 16), jnp.float32)

    agg1, cnt = _sc_seg_sum(x[:, :64], x[:, 64:], srcp, dstp, zrows, zcnt,
                            ones_r, True)
    craw = cnt[:_N]

    hq = _phase_a(
        x, agg1[0, :_N], agg1[1, :_N], craw, Wl1.T, bl1.reshape(1, -1),
        Wr1.T, g1.reshape(1, -1), b1.reshape(1, -1))

    agg2f, _ = _sc_seg_sum(hq[0], hq[1], srcp, dstp, zrows, zcnt, ones_r, False)
    agg2b, _ = _sc_seg_sum(hq[2], hq[3], srcp, dstp, zrows, zcnt, ones_r, False)

    yla, ylb, yr = _phase_b(
        hq, agg2f[0, :_N], agg2f[1, :_N], agg2b[0, :_N], agg2b[1, :_N], craw,
        Wl2.T, bl2.reshape(1, -1), Wr2.T, g2.reshape(1, -1), b2.reshape(1, -1),
        Wl3.T, Wr3.T)

    agg3, _ = _sc_seg_sum(yla, ylb, srcp, dstp, zrows, zcnt, ones_r, False)

    z, node_logits, zu_proj, zv_proj = _phase_c(
        agg3[0, :_N], agg3[1, :_N], craw, yr, bl3.reshape(1, -1), Wc1.T,
        bc1.reshape(1, -1), Wc2.T, bc2.reshape(1, -1), Wp1[:, :128].T,
        Wp1[:, 128:].T, bp1.reshape(1, -1))

    u = jnp.concatenate([pos_edge_index[0], neg_edge_index[0]])
    v = jnp.concatenate([pos_edge_index[1], neg_edge_index[1]])
    gu, gv = _sc_link_gather(zu_proj, zv_proj, u, v)

    link_probs = _phase_d(gu, gv, Wp2, bp2.reshape(1, 1))
    return (z, node_logits, link_probs)


# speedup vs baseline: 5.0002x; 1.3645x over previous
import numpy as np
import jax
from jax import lax
import jax.numpy as jnp
from jax.experimental import pallas as pl
from jax.experimental.pallas import tpu as pltpu
from jax.experimental.pallas import tpu_sc as plsc

_N = 10000
_E = 320000
_P2 = 131072
_BN_S = np.float32(1.0 / np.sqrt(1.0 + 1e-5))

_NC, _NS = 2, 16
_NPAD = 10240
_EPAD = 327680
_CHUNK = 256
_RPT = _NPAD // _NS

_MESH = plsc.VectorSubcoreMesh(core_axis_name="c", subcore_axis_name="s",
                               num_cores=_NC, num_subcores=_NS)


_NCHUNKS = _EPAD // _NS // _CHUNK
_NPAIRS = _NCHUNKS // 2


def _sc_seg_sum(ta, tb, srcp, dstp, zrows, zcnt, ones_r, with_cnt):
    def body(tar, tbr, srcr, dstr, zr, zc, onesr, out, outc,
             idxs, idxd, rows_a, rows_b, ones_v, acc, accc,
             semi, sem_a, sem_b):
        c = lax.axis_index("c")
        s = lax.axis_index("s")
        r0 = s * _RPT
        ci = pltpu.async_copy(srcr.at[s], idxs, semi)
        cd = pltpu.async_copy(dstr.at[s], idxd, semi)
        pltpu.sync_copy(zr, acc.at[pl.ds(r0, _RPT)])
        if with_cnt:
            @pl.when(c == 0)
            def _():
                pltpu.sync_copy(zc, accc.at[pl.ds(r0, _RPT)])
                pltpu.sync_copy(onesr, ones_v)
        ci.wait()
        cd.wait()

        def gather(k, rows, sem):
            @pl.when(c == 0)
            def _():
                pltpu.async_copy(tar.at[idxs.at[k]], rows, sem)

            @pl.when(c == 1)
            def _():
                pltpu.async_copy(tbr.at[idxs.at[k]], rows, sem)

        def consume(k, rows, sem):
            pltpu.make_async_copy(tar.at[idxs.at[k]], rows, sem).wait()
            pltpu.sync_copy(rows, acc.at[idxd.at[k]], add=True)
            if with_cnt:
                @pl.when(c == 0)
                def _():
                    pltpu.sync_copy(ones_v, accc.at[idxd.at[k]], add=True)

        gather(0, rows_a, sem_a)
        plsc.subcore_barrier()

        @pl.loop(0, _NPAIRS)
        def _(p):
            ka = 2 * p
            gather(ka + 1, rows_b, sem_b)
            consume(ka, rows_a, sem_a)

            @pl.when(p + 1 < _NPAIRS)
            def _():
                gather(ka + 2, rows_a, sem_a)

            consume(ka + 1, rows_b, sem_b)

        plsc.subcore_barrier()
        pltpu.sync_copy(acc.at[pl.ds(r0, _RPT)], out.at[c, pl.ds(r0, _RPT)])
        if with_cnt:
            @pl.when(c == 0)
            def _():
                pltpu.sync_copy(accc.at[pl.ds(r0, _RPT)], outc.at[pl.ds(r0, _RPT)])

    f = pl.kernel(
        body,
        out_type=(jax.ShapeDtypeStruct((_NC, _NPAD, 64), jnp.float32),
                  jax.ShapeDtypeStruct((_NPAD, 16), jnp.float32)),
        mesh=_MESH,
        compiler_params=pltpu.CompilerParams(use_tc_tiling_on_sc=False),
        scratch_types=[
            pltpu.VMEM((_NCHUNKS, _CHUNK), jnp.int32),
            pltpu.VMEM((_NCHUNKS, _CHUNK), jnp.int32),
            pltpu.VMEM((_CHUNK, 64), jnp.float32),
            pltpu.VMEM((_CHUNK, 64), jnp.float32),
            pltpu.VMEM((_CHUNK, 16), jnp.float32),
            pltpu.VMEM_SHARED((_NPAD, 64), jnp.float32),
            pltpu.VMEM_SHARED((_NPAD, 16), jnp.float32),
            pltpu.SemaphoreType.DMA,
            pltpu.SemaphoreType.DMA,
            pltpu.SemaphoreType.DMA,
        ],
    )
    return f(ta, tb, srcp, dstp, zrows, zcnt, ones_r)


def _sc_link_gather(zu, zv, uidx, vidx):
    ept = _P2 // (_NC * _NS)
    nch = ept // _CHUNK

    def body(tu, tv, ur, vr, gu, gv, idx_u, idx_v, rows_u, rows_v, sem):
        c = lax.axis_index("c")
        s = lax.axis_index("s")
        base = (c * _NS + s) * ept

        @pl.loop(0, nch)
        def _(k):
            off = base + k * _CHUNK
            pltpu.sync_copy(ur.at[pl.ds(off, _CHUNK)], idx_u)
            pltpu.sync_copy(vr.at[pl.ds(off, _CHUNK)], idx_v)
            pltpu.async_copy(tu.at[idx_u], rows_u, sem).wait()
            pltpu.sync_copy(rows_u, gu.at[pl.ds(off, _CHUNK)])
            pltpu.async_copy(tv.at[idx_v], rows_v, sem).wait()
            pltpu.sync_copy(rows_v, gv.at[pl.ds(off, _CHUNK)])

    f = pl.kernel(
        body,
        out_type=(jax.ShapeDtypeStruct((_P2, 128), jnp.float32),
                  jax.ShapeDtypeStruct((_P2, 128), jnp.float32)),
        mesh=_MESH,
        scratch_types=[
            pltpu.VMEM((_CHUNK,), jnp.int32),
            pltpu.VMEM((_CHUNK,), jnp.int32),
            pltpu.VMEM((_CHUNK, 128), jnp.float32),
            pltpu.VMEM((_CHUNK, 128), jnp.float32),
            pltpu.SemaphoreType.DMA,
        ],
    )
    return f(zu, zv, uidx, vidx)


def _cnt_col(craw):
    return jnp.maximum(craw[:, :1], 1.0)


_RB = 2000
_NRB = _N // _RB


def _row_call(body, n_rowed, weight_shapes, out_cols):
    def block(c):
        return pl.BlockSpec((_RB, c), lambda i: (i, 0))

    def full(shape):
        return pl.BlockSpec(shape, lambda i: (0, 0))

    def make(*arrays):
        in_specs = [block(a.shape[1]) for a in arrays[:n_rowed]]
        in_specs += [full(s) for s in weight_shapes]
        return pl.pallas_call(
            body,
            grid=(_NRB,),
            in_specs=in_specs,
            out_specs=tuple(block(c) for c in out_cols),
            out_shape=tuple(jax.ShapeDtypeStruct((_N, c), jnp.float32)
                            for c in out_cols),
        )(*arrays)
    return make


def _phase_a(x, a0, a1, craw, Wl1T, bl1, Wr1T, g1, b1):
    def body(x_ref, a0r, a1r, cr, wl_ref, blr, wr_ref, gr, br, h0, h1r, h2, h3):
        agg = jnp.concatenate([a0r[...], a1r[...]], axis=-1)
        mean = agg / _cnt_col(cr[...])
        h = jnp.dot(mean, wl_ref[...], preferred_element_type=jnp.float32)
        h = h + jnp.dot(x_ref[...], wr_ref[...], preferred_element_type=jnp.float32)
        h = (h + blr[...]) * (gr[...] * _BN_S) + br[...]
        h = jnp.maximum(h, 0.0)
        h0[...] = h[:, 0:64]
        h1r[...] = h[:, 64:128]
        h2[...] = h[:, 128:192]
        h3[...] = h[:, 192:256]

    wts = [Wl1T, bl1, Wr1T, g1, b1]
    return _row_call(body, 4, [w.shape for w in wts], (64, 64, 64, 64))(
        x, a0, a1, craw, *wts)


def _phase_b(hq, a0, a1, a2, a3, craw, Wl2T, bl2, Wr2T, g2, b2, Wl3T, Wr3T):
    def body(h0, h1r, h2r, h3, a0r, a1r, a2r, a3r, cr, wl, blr, wr, gr, br,
             wl3, wr3, yla_ref, ylb_ref, yr_ref):
        cnt = _cnt_col(cr[...])
        mean = jnp.concatenate(
            [a0r[...], a1r[...], a2r[...], a3r[...]], axis=-1) / cnt
        h1 = jnp.concatenate([h0[...], h1r[...], h2r[...], h3[...]], axis=-1)
        h = jnp.dot(mean, wl[...], preferred_element_type=jnp.float32)
        h = h + jnp.dot(h1, wr[...], preferred_element_type=jnp.float32)
        h = (h + blr[...]) * (gr[...] * _BN_S) + br[...]
        h = jnp.maximum(h, 0.0)
        yl = jnp.dot(h, wl3[...], preferred_element_type=jnp.float32)
        yla_ref[...] = yl[:, :64]
        ylb_ref[...] = yl[:, 64:]
        yr_ref[...] = jnp.dot(h, wr3[...], preferred_element_type=jnp.float32)

    wts = [Wl2T, bl2, Wr2T, g2, b2, Wl3T, Wr3T]
    return _row_call(body, 9, [w.shape for w in wts], (64, 64, 128))(
        *hq, a0, a1, a2, a3, craw, *wts)


def _phase_c(a0, a1, craw, yr, bl3, Wc1T, bc1, Wc2T, bc2, Wp1aT, Wp1bT, bp1):
    def body(a0r, a1r, cr, yr_ref, bl3r, wc1, bc1r, wc2, bc2r, wpa, wpb,
             bp1r, z_ref, nl_ref, zu_ref, zv_ref):
        agg = jnp.concatenate([a0r[...], a1r[...]], axis=-1)
        z = agg / _cnt_col(cr[...]) + bl3r[...] + yr_ref[...]
        z_ref[...] = z
        t = jnp.maximum(jnp.dot(z, wc1[...], preferred_element_type=jnp.float32)
                        + bc1r[...], 0.0)
        nl_ref[...] = jnp.dot(t, wc2[...], preferred_element_type=jnp.float32) + bc2r[...]
        zu_ref[...] = jnp.dot(z, wpa[...], preferred_element_type=jnp.float32) + bp1r[...]
        zv_ref[...] = jnp.dot(z, wpb[...], preferred_element_type=jnp.float32)

    wts = [bl3, Wc1T, bc1, Wc2T, bc2, Wp1aT, Wp1bT, bp1]
    return _row_call(body, 4, [w.shape for w in wts], (128, 4, 128, 128))(
        a0, a1, craw, yr, *wts)


def _phase_d(gu, gv, wp2, bp2):
    _NB = 16
    _B = _P2 // _NB

    def body(gu_ref, gv_ref, w_ref, b_ref, out_ref):
        e = jnp.maximum(gu_ref[...] + gv_ref[...], 0.0)
        logit = jnp.sum(e * w_ref[...], axis=-1) + b_ref[0, 0]
        out_ref[...] = jax.nn.sigmoid(logit).reshape(_B // 128, 128)

    out = pl.pallas_call(
        body,
        grid=(_NB,),
        in_specs=[pl.BlockSpec((_B, 128), lambda i: (i, 0)),
                  pl.BlockSpec((_B, 128), lambda i: (i, 0)),
                  pl.BlockSpec((1, 128), lambda i: (0, 0)),
                  pl.BlockSpec((1, 1), lambda i: (0, 0))],
        out_specs=pl.BlockSpec((_B // 128, 128), lambda i: (i, 0)),
        out_shape=jax.ShapeDtypeStruct((_P2 // 128, 128), jnp.float32),
    )(gu, gv, wp2, bp2)
    return out.reshape(_P2)


def kernel(x, edge_index, pos_edge_index, neg_edge_index, Wl1, bl1, Wr1, g1, b1,
           Wl2, bl2, Wr2, g2, b2, Wl3, bl3, Wr3, Wp1, bp1, Wp2, bp2, Wc1, bc1,
           Wc2, bc2):
    src = edge_index[0]
    dst = edge_index[1]
    srcp = jnp.concatenate([src, jnp.zeros((_EPAD - _E,), jnp.int32)])
    dstp = jnp.concatenate([dst, jnp.full((_EPAD - _E,), _NPAD - 1, jnp.int32)])
    srcp = srcp.reshape(_NS, _NCHUNKS, _CHUNK)
    dstp = dstp.reshape(_NS, _NCHUNKS, _CHUNK)

    zrows = jnp.zeros((_RPT, 64), jnp.float32)
    zcnt = jnp.zeros((_RPT, 16), jnp.float32)
    ones_r = jnp.ones((_CHUNK, 16), jnp.float32)

    agg1, cnt = _sc_seg_sum(x[:, :64], x[:, 64:], srcp, dstp, zrows, zcnt,
                            ones_r, True)
    craw = cnt[:_N]

    hq = _phase_a(
        x, agg1[0, :_N], agg1[1, :_N], craw, Wl1.T, bl1.reshape(1, -1),
        Wr1.T, g1.reshape(1, -1), b1.reshape(1, -1))

    agg2f, _ = _sc_seg_sum(hq[0], hq[1], srcp, dstp, zrows, zcnt, ones_r, False)
    agg2b, _ = _sc_seg_sum(hq[2], hq[3], srcp, dstp, zrows, zcnt, ones_r, False)

    yla, ylb, yr = _phase_b(
        hq, agg2f[0, :_N], agg2f[1, :_N], agg2b[0, :_N], agg2b[1, :_N], craw,
        Wl2.T, bl2.reshape(1, -1), Wr2.T, g2.reshape(1, -1), b2.reshape(1, -1),
        Wl3.T, Wr3.T)

    agg3, _ = _sc_seg_sum(yla, ylb, srcp, dstp, zrows, zcnt, ones_r, False)

    z, node_logits, zu_proj, zv_proj = _phase_c(
        agg3[0, :_N], agg3[1, :_N], craw, yr, bl3.reshape(1, -1), Wc1.T,
        bc1.reshape(1, -1), Wc2.T, bc2.reshape(1, -1), Wp1[:, :128].T,
        Wp1[:, 128:].T, bp1.reshape(1, -1))

    u = jnp.concatenate([pos_edge_index[0], neg_edge_index[0]])
    v = jnp.concatenate([pos_edge_index[1], neg_edge_index[1]])
    gu, gv = _sc_link_gather(zu_proj, zv_proj, u, v)

    link_probs = _phase_d(gu, gv, Wp2, bp2.reshape(1, 1))
    return (z, node_logits, link_probs)


# speedup vs baseline: 5.3323x; 1.0664x over previous
import numpy as np
import jax
from jax import lax
import jax.numpy as jnp
from jax.experimental import pallas as pl
from jax.experimental.pallas import tpu as pltpu
from jax.experimental.pallas import tpu_sc as plsc

_N = 10000
_E = 320000
_P2 = 131072
_BN_S = np.float32(1.0 / np.sqrt(1.0 + 1e-5))

_NC, _NS = 2, 16
_NPAD = 10240
_EPAD = 327680
_CHUNK = 256
_RPT = _NPAD // _NS

_MESH = plsc.VectorSubcoreMesh(core_axis_name="c", subcore_axis_name="s",
                               num_cores=_NC, num_subcores=_NS)


_NCHUNKS = _EPAD // _NS // _CHUNK
_NPAIRS = _NCHUNKS // 2


def _sc_seg_sum(ta, tb, srcp, dstp, zrows, zcnt, ones_r, with_cnt):
    def body(tar, tbr, srcr, dstr, zr, zc, onesr, out, outc,
             idxs, idxd, rows_a, rows_b, ones_v, acc, accc,
             semi, sem_a, sem_b):
        c = lax.axis_index("c")
        s = lax.axis_index("s")
        r0 = s * _RPT
        ci = pltpu.async_copy(srcr.at[s], idxs, semi)
        cd = pltpu.async_copy(dstr.at[s], idxd, semi)
        pltpu.sync_copy(zr, acc.at[pl.ds(r0, _RPT)])
        if with_cnt:
            @pl.when(c == 0)
            def _():
                pltpu.sync_copy(zc, accc.at[pl.ds(r0, _RPT)])
                pltpu.sync_copy(onesr, ones_v)
        ci.wait()
        cd.wait()

        def gather(k, rows, sem):
            @pl.when(c == 0)
            def _():
                pltpu.async_copy(tar.at[idxs.at[k]], rows, sem)

            @pl.when(c == 1)
            def _():
                pltpu.async_copy(tbr.at[idxs.at[k]], rows, sem)

        def consume(k, rows, sem):
            pltpu.make_async_copy(tar.at[idxs.at[k]], rows, sem).wait()
            pltpu.sync_copy(rows, acc.at[idxd.at[k]], add=True)
            if with_cnt:
                @pl.when(c == 0)
                def _():
                    pltpu.sync_copy(ones_v, accc.at[idxd.at[k]], add=True)

        gather(0, rows_a, sem_a)
        plsc.subcore_barrier()

        @pl.loop(0, _NPAIRS)
        def _(p):
            ka = 2 * p
            gather(ka + 1, rows_b, sem_b)
            consume(ka, rows_a, sem_a)

            @pl.when(p + 1 < _NPAIRS)
            def _():
                gather(ka + 2, rows_a, sem_a)

            consume(ka + 1, rows_b, sem_b)

        plsc.subcore_barrier()
        pltpu.sync_copy(acc.at[pl.ds(r0, _RPT)], out.at[c, pl.ds(r0, _RPT)])
        if with_cnt:
            @pl.when(c == 0)
            def _():
                pltpu.sync_copy(accc.at[pl.ds(r0, _RPT)], outc.at[pl.ds(r0, _RPT)])

    f = pl.kernel(
        body,
        out_type=(jax.ShapeDtypeStruct((_NC, _NPAD, 64), jnp.float32),
                  jax.ShapeDtypeStruct((_NPAD, 16), jnp.float32)),
        mesh=_MESH,
        compiler_params=pltpu.CompilerParams(use_tc_tiling_on_sc=False),
        scratch_types=[
            pltpu.VMEM((_NCHUNKS, _CHUNK), jnp.int32),
            pltpu.VMEM((_NCHUNKS, _CHUNK), jnp.int32),
            pltpu.VMEM((_CHUNK, 64), jnp.float32),
            pltpu.VMEM((_CHUNK, 64), jnp.float32),
            pltpu.VMEM((_CHUNK, 16), jnp.float32),
            pltpu.VMEM_SHARED((_NPAD, 64), jnp.float32),
            pltpu.VMEM_SHARED((_NPAD, 16), jnp.float32),
            pltpu.SemaphoreType.DMA,
            pltpu.SemaphoreType.DMA,
            pltpu.SemaphoreType.DMA,
        ],
    )
    return f(ta, tb, srcp, dstp, zrows, zcnt, ones_r)


_EPT_L = _P2 // (_NC * _NS)
_NCH_L = _EPT_L // _CHUNK


def _sc_link_gather(zu, zv, uidx, vidx):
    def body(tu, tv, ur, vr, gu, gv, idx_u, idx_v, rows_u, rows_v,
             semi, sem_u, sem_v):
        c = lax.axis_index("c")
        s = lax.axis_index("s")
        t = c * _NS + s
        base = t * _EPT_L
        cu = pltpu.async_copy(ur.at[t], idx_u, semi)
        cv = pltpu.async_copy(vr.at[t], idx_v, semi)
        cu.wait()
        cv.wait()
        pltpu.async_copy(tu.at[idx_u.at[0]], rows_u, sem_u)

        @pl.loop(0, _NCH_L)
        def _(k):
            off = base + k * _CHUNK
            pltpu.async_copy(tv.at[idx_v.at[k]], rows_v, sem_v)
            pltpu.make_async_copy(tu.at[idx_u.at[k]], rows_u, sem_u).wait()
            pltpu.sync_copy(rows_u, gu.at[pl.ds(off, _CHUNK)])

            @pl.when(k + 1 < _NCH_L)
            def _():
                pltpu.async_copy(tu.at[idx_u.at[k + 1]], rows_u, sem_u)

            pltpu.make_async_copy(tv.at[idx_v.at[k]], rows_v, sem_v).wait()
            pltpu.sync_copy(rows_v, gv.at[pl.ds(off, _CHUNK)])

    f = pl.kernel(
        body,
        out_type=(jax.ShapeDtypeStruct((_P2, 128), jnp.float32),
                  jax.ShapeDtypeStruct((_P2, 128), jnp.float32)),
        mesh=_MESH,
        compiler_params=pltpu.CompilerParams(use_tc_tiling_on_sc=False),
        scratch_types=[
            pltpu.VMEM((_NCH_L, _CHUNK), jnp.int32),
            pltpu.VMEM((_NCH_L, _CHUNK), jnp.int32),
            pltpu.VMEM((_CHUNK, 128), jnp.float32),
            pltpu.VMEM((_CHUNK, 128), jnp.float32),
            pltpu.SemaphoreType.DMA,
            pltpu.SemaphoreType.DMA,
            pltpu.SemaphoreType.DMA,
        ],
    )
    return f(zu, zv, uidx, vidx)


def _cnt_col(craw):
    return jnp.maximum(craw[:, :1], 1.0)


_RB = 2000
_NRB = _N // _RB


def _row_call(body, n_rowed, weight_shapes, out_cols):
    def block(c):
        return pl.BlockSpec((_RB, c), lambda i: (i, 0))

    def full(shape):
        return pl.BlockSpec(shape, lambda i: (0, 0))

    def make(*arrays):
        in_specs = [block(a.shape[1]) for a in arrays[:n_rowed]]
        in_specs += [full(s) for s in weight_shapes]
        return pl.pallas_call(
            body,
            grid=(_NRB,),
            in_specs=in_specs,
            out_specs=tuple(block(c) for c in out_cols),
            out_shape=tuple(jax.ShapeDtypeStruct((_N, c), jnp.float32)
                            for c in out_cols),
        )(*arrays)
    return make


def _phase_a(x, a0, a1, craw, Wl1T, bl1, Wr1T, g1, b1):
    def body(x_ref, a0r, a1r, cr, wl_ref, blr, wr_ref, gr, br, h0, h1r, h2, h3):
        agg = jnp.concatenate([a0r[...], a1r[...]], axis=-1)
        mean = agg / _cnt_col(cr[...])
        h = jnp.dot(mean, wl_ref[...], preferred_element_type=jnp.float32)
        h = h + jnp.dot(x_ref[...], wr_ref[...], preferred_element_type=jnp.float32)
        h = (h + blr[...]) * (gr[...] * _BN_S) + br[...]
        h = jnp.maximum(h, 0.0)
        h0[...] = h[:, 0:64]
        h1r[...] = h[:, 64:128]
        h2[...] = h[:, 128:192]
        h3[...] = h[:, 192:256]

    wts = [Wl1T, bl1, Wr1T, g1, b1]
    return _row_call(body, 4, [w.shape for w in wts], (64, 64, 64, 64))(
        x, a0, a1, craw, *wts)


def _phase_b(hq, a0, a1, a2, a3, craw, Wl2T, bl2, Wr2T, g2, b2, Wl3T, Wr3T):
    def body(h0, h1r, h2r, h3, a0r, a1r, a2r, a3r, cr, wl, blr, wr, gr, br,
             wl3, wr3, yla_ref, ylb_ref, yr_ref):
        cnt = _cnt_col(cr[...])
        mean = jnp.concatenate(
            [a0r[...], a1r[...], a2r[...], a3r[...]], axis=-1) / cnt
        h1 = jnp.concatenate([h0[...], h1r[...], h2r[...], h3[...]], axis=-1)
        h = jnp.dot(mean, wl[...], preferred_element_type=jnp.float32)
        h = h + jnp.dot(h1, wr[...], preferred_element_type=jnp.float32)
        h = (h + blr[...]) * (gr[...] * _BN_S) + br[...]
        h = jnp.maximum(h, 0.0)
        yl = jnp.dot(h, wl3[...], preferred_element_type=jnp.float32)
        yla_ref[...] = yl[:, :64]
        ylb_ref[...] = yl[:, 64:]
        yr_ref[...] = jnp.dot(h, wr3[...], preferred_element_type=jnp.float32)

    wts = [Wl2T, bl2, Wr2T, g2, b2, Wl3T, Wr3T]
    return _row_call(body, 9, [w.shape for w in wts], (64, 64, 128))(
        *hq, a0, a1, a2, a3, craw, *wts)


def _phase_c(a0, a1, craw, yr, bl3, Wc1T, bc1, Wc2T, bc2, Wp1aT, Wp1bT, bp1):
    def body(a0r, a1r, cr, yr_ref, bl3r, wc1, bc1r, wc2, bc2r, wpa, wpb,
             bp1r, z_ref, nl_ref, zu_ref, zv_ref):
        agg = jnp.concatenate([a0r[...], a1r[...]], axis=-1)
        z = agg / _cnt_col(cr[...]) + bl3r[...] + yr_ref[...]
        z_ref[...] = z
        t = jnp.maximum(jnp.dot(z, wc1[...], preferred_element_type=jnp.float32)
                        + bc1r[...], 0.0)
        nl_ref[...] = jnp.dot(t, wc2[...], preferred_element_type=jnp.float32) + bc2r[...]
        zu_ref[...] = jnp.dot(z, wpa[...], preferred_element_type=jnp.float32) + bp1r[...]
        zv_ref[...] = jnp.dot(z, wpb[...], preferred_element_type=jnp.float32)

    wts = [bl3, Wc1T, bc1, Wc2T, bc2, Wp1aT, Wp1bT, bp1]
    return _row_call(body, 4, [w.shape for w in wts], (128, 4, 128, 128))(
        a0, a1, craw, yr, *wts)


def _phase_d(gu, gv, wp2, bp2):
    _NB = 16
    _B = _P2 // _NB

    def body(gu_ref, gv_ref, w_ref, b_ref, out_ref):
        e = jnp.maximum(gu_ref[...] + gv_ref[...], 0.0)
        logit = jnp.sum(e * w_ref[...], axis=-1) + b_ref[0, 0]
        out_ref[...] = jax.nn.sigmoid(logit).reshape(_B // 128, 128)

    out = pl.pallas_call(
        body,
        grid=(_NB,),
        in_specs=[pl.BlockSpec((_B, 128), lambda i: (i, 0)),
                  pl.BlockSpec((_B, 128), lambda i: (i, 0)),
                  pl.BlockSpec((1, 128), lambda i: (0, 0)),
                  pl.BlockSpec((1, 1), lambda i: (0, 0))],
        out_specs=pl.BlockSpec((_B // 128, 128), lambda i: (i, 0)),
        out_shape=jax.ShapeDtypeStruct((_P2 // 128, 128), jnp.float32),
    )(gu, gv, wp2, bp2)
    return out.reshape(_P2)


def kernel(x, edge_index, pos_edge_index, neg_edge_index, Wl1, bl1, Wr1, g1, b1,
           Wl2, bl2, Wr2, g2, b2, Wl3, bl3, Wr3, Wp1, bp1, Wp2, bp2, Wc1, bc1,
           Wc2, bc2):
    src = edge_index[0]
    dst = edge_index[1]
    srcp = jnp.concatenate([src, jnp.zeros((_EPAD - _E,), jnp.int32)])
    dstp = jnp.concatenate([dst, jnp.full((_EPAD - _E,), _NPAD - 1, jnp.int32)])
    srcp = srcp.reshape(_NS, _NCHUNKS, _CHUNK)
    dstp = dstp.reshape(_NS, _NCHUNKS, _CHUNK)

    zrows = jnp.zeros((_RPT, 64), jnp.float32)
    zcnt = jnp.zeros((_RPT, 16), jnp.float32)
    ones_r = jnp.ones((_CHUNK, 16), jnp.float32)

    agg1, cnt = _sc_seg_sum(x[:, :64], x[:, 64:], srcp, dstp, zrows, zcnt,
                            ones_r, True)
    craw = cnt[:_N]

    hq = _phase_a(
        x, agg1[0, :_N], agg1[1, :_N], craw, Wl1.T, bl1.reshape(1, -1),
        Wr1.T, g1.reshape(1, -1), b1.reshape(1, -1))

    agg2f, _ = _sc_seg_sum(hq[0], hq[1], srcp, dstp, zrows, zcnt, ones_r, False)
    agg2b, _ = _sc_seg_sum(hq[2], hq[3], srcp, dstp, zrows, zcnt, ones_r, False)

    yla, ylb, yr = _phase_b(
        hq, agg2f[0, :_N], agg2f[1, :_N], agg2b[0, :_N], agg2b[1, :_N], craw,
        Wl2.T, bl2.reshape(1, -1), Wr2.T, g2.reshape(1, -1), b2.reshape(1, -1),
        Wl3.T, Wr3.T)

    agg3, _ = _sc_seg_sum(yla, ylb, srcp, dstp, zrows, zcnt, ones_r, False)

    z, node_logits, zu_proj, zv_proj = _phase_c(
        agg3[0, :_N], agg3[1, :_N], craw, yr, bl3.reshape(1, -1), Wc1.T,
        bc1.reshape(1, -1), Wc2.T, bc2.reshape(1, -1), Wp1[:, :128].T,
        Wp1[:, 128:].T, bp1.reshape(1, -1))

    u = jnp.concatenate([pos_edge_index[0], neg_edge_index[0]])
    v = jnp.concatenate([pos_edge_index[1], neg_edge_index[1]])
    u = u.reshape(_NC * _NS, _NCH_L, _CHUNK)
    v = v.reshape(_NC * _NS, _NCH_L, _CHUNK)
    gu, gv = _sc_link_gather(zu_proj, zv_proj, u, v)

    link_probs = _phase_d(gu, gv, Wp2, bp2.reshape(1, 1))
    return (z, node_logits, link_probs)
